# baseline (speedup 1.0000x reference)
import functools
import jax
import jax.numpy as jnp
from jax import lax
from jax.experimental import pallas as pl
from jax.experimental.pallas import tpu as pltpu
from jax.experimental.pallas import tpu_sc as plsc

N_NODES = 10000
D = 128
E = 320000
NW = 32
TILES = 16
CHUNK = 128
CHUNKS = 80
STAGE = 40
E_PAD = NW * CHUNKS * CHUNK
N_PAD = 10240
ROWS_PER_TILE = N_PAD // TILES

_mesh = plsc.VectorSubcoreMesh(core_axis_name="c", subcore_axis_name="s")


@functools.partial(
    pl.kernel,
    out_type=jax.ShapeDtypeStruct((NW, N_PAD), jnp.float32),
    mesh=_mesh,
    compiler_params=pltpu.CompilerParams(needs_layout_passes=False),
    scratch_types=[
        pltpu.VMEM((CHUNKS, CHUNK), jnp.int32),
        pltpu.VMEM((N_PAD,), jnp.float32),
    ],
)
def _deg_kernel(dst_hbm, out_hbm, idx_v, deg_v):
    cid = lax.axis_index("c")
    sid = lax.axis_index("s")
    wid = cid * TILES + sid
    for q in range(CHUNKS // STAGE):
        pltpu.sync_copy(dst_hbm.at[wid, q],
                        idx_v.at[pl.ds(q * STAGE, STAGE)])
    z16 = jnp.zeros((16,), jnp.float32)

    def zbody(i, _):
        deg_v[pl.ds(i * 16, 16)] = z16
        return 0

    lax.fori_loop(0, N_PAD // 16, zbody, 0)
    ones16 = jnp.ones((16,), jnp.float32)

    def body(j, _):
        for k in range(CHUNK // 16):
            idx16 = idx_v[j, pl.ds(k * 16, 16)]
            plsc.addupdate_scatter(deg_v, [idx16], ones16)
        return 0

    lax.fori_loop(0, CHUNKS, body, 0)
    pltpu.sync_copy(deg_v, out_hbm.at[wid])


@functools.partial(
    pl.kernel,
    out_type=jax.ShapeDtypeStruct((2, N_PAD, D), jnp.float32),
    mesh=_mesh,
    scratch_types=[
        pltpu.VMEM((CHUNKS, CHUNK), jnp.int32),
        pltpu.VMEM((CHUNKS, CHUNK), jnp.int32),
        pltpu.VMEM((CHUNK, D), jnp.float32),
        pltpu.VMEM_SHARED((N_PAD, D), jnp.float32),
        pltpu.SemaphoreType.DMA,
    ],
)
def _agg_kernel(xs_hbm, src_hbm, dst_hbm, zero_hbm, out_hbm,
                src_v, dst_v, rows0_v, acc_sh, sem0):
    cid = lax.axis_index("c")
    sid = lax.axis_index("s")
    wid = cid * TILES + sid
    pltpu.sync_copy(zero_hbm.at[pl.ds(sid * ROWS_PER_TILE, ROWS_PER_TILE)],
                    acc_sh.at[pl.ds(sid * ROWS_PER_TILE, ROWS_PER_TILE)])
    plsc.subcore_barrier()

    for q in range(CHUNKS // STAGE):
        pltpu.sync_copy(src_hbm.at[wid, q],
                        src_v.at[pl.ds(q * STAGE, STAGE)])
        pltpu.sync_copy(dst_hbm.at[wid, q],
                        dst_v.at[pl.ds(q * STAGE, STAGE)])

    def body(j, _):
        pltpu.async_copy(xs_hbm.at[src_v.at[j]], rows0_v, sem0).wait()
        pltpu.sync_copy(rows0_v, acc_sh.at[dst_v.at[j]], add=True)
        return 0

    lax.fori_loop(0, CHUNKS, body, 0)
    plsc.subcore_barrier()
    pltpu.sync_copy(acc_sh.at[pl.ds(sid * ROWS_PER_TILE, ROWS_PER_TILE)],
                    out_hbm.at[cid, pl.ds(sid * ROWS_PER_TILE, ROWS_PER_TILE)])


def _tc1_body(x_ref, w_ref, degp_ref, xs_ref, dinv_ref):
    deg = jnp.sum(degp_ref[...], axis=0) + 1.0
    dinv = lax.rsqrt(deg)
    dinv_ref[...] = dinv
    xw = jnp.dot(x_ref[...], w_ref[...], preferred_element_type=jnp.float32)
    xs_ref[...] = xw * dinv[:N_NODES, None]


def _tc2_body(p_ref, xs_ref, dinv_ref, b_ref, w_ref, xs2_ref):
    dinv = dinv_ref[...][:N_NODES, None]
    acc = p_ref[0, :N_NODES, :] + p_ref[1, :N_NODES, :] + xs_ref[...]
    h = jnp.maximum(dinv * acc + b_ref[...][None, :], 0.0)
    xs2_ref[...] = jnp.dot(h, w_ref[...], preferred_element_type=jnp.float32) * dinv


def _tc3_body(p_ref, xs_ref, dinv_ref, b_ref, out_ref):
    dinv = dinv_ref[...][:N_NODES, None]
    acc = p_ref[0, :N_NODES, :] + p_ref[1, :N_NODES, :] + xs_ref[...]
    out_ref[...] = dinv * acc + b_ref[...][None, :]


def _tc_call(body, out_shape, *args):
    return pl.pallas_call(body, out_shape=out_shape)(*args)


def kernel(x, edge_index, W1, b1, W2, b2):
    ei = edge_index.astype(jnp.int32)
    pad = E_PAD - E
    src = jnp.concatenate([ei[0], jnp.zeros((pad,), jnp.int32)])
    junk = N_NODES + (jnp.arange(pad, dtype=jnp.int32) % (N_PAD - N_NODES))
    dst = jnp.concatenate([ei[1], junk])
    src3 = src.reshape(NW, CHUNKS // STAGE, STAGE, CHUNK)
    dst3 = dst.reshape(NW, CHUNKS // STAGE, STAGE, CHUNK)
    zeros = jnp.zeros((N_PAD, D), jnp.float32)

    degp = _deg_kernel(dst3)
    xs1, dinv = _tc_call(
        _tc1_body,
        (jax.ShapeDtypeStruct((N_NODES, D), jnp.float32),
         jax.ShapeDtypeStruct((N_PAD,), jnp.float32)),
        x, W1, degp)
    parts1 = _agg_kernel(xs1, src3, dst3, zeros)
    xs2 = _tc_call(
        _tc2_body,
        jax.ShapeDtypeStruct((N_NODES, D), jnp.float32),
        parts1, xs1, dinv, b1, W2)
    parts2 = _agg_kernel(xs2, src3, dst3, zeros)
    out = _tc_call(
        _tc3_body,
        jax.ShapeDtypeStruct((N_NODES, D), jnp.float32),
        parts2, xs2, dinv, b2)
    return out


# speedup vs baseline: 2.4670x; 2.4670x over previous
import functools
import jax
import jax.numpy as jnp
from jax import lax
from jax.experimental import pallas as pl
from jax.experimental.pallas import tpu as pltpu
from jax.experimental.pallas import tpu_sc as plsc

N_NODES = 10000
D = 128
E = 320000
NW = 32
TILES = 16
CHUNK = 128
CHUNKS = 80
STAGE = 40
E_PAD = NW * CHUNKS * CHUNK
N_PAD = 10240
ROWS_PER_TILE = N_PAD // TILES

_mesh = plsc.VectorSubcoreMesh(core_axis_name="c", subcore_axis_name="s")


@functools.partial(
    pl.kernel,
    out_type=jax.ShapeDtypeStruct((NW, N_PAD), jnp.float32),
    mesh=_mesh,
    compiler_params=pltpu.CompilerParams(needs_layout_passes=False),
    scratch_types=[
        pltpu.VMEM((CHUNKS, CHUNK), jnp.int32),
        pltpu.VMEM((N_PAD,), jnp.float32),
    ],
)
def _deg_kernel(dst_hbm, out_hbm, idx_v, deg_v):
    cid = lax.axis_index("c")
    sid = lax.axis_index("s")
    wid = cid * TILES + sid
    for q in range(CHUNKS // STAGE):
        pltpu.sync_copy(dst_hbm.at[wid, q],
                        idx_v.at[pl.ds(q * STAGE, STAGE)])
    z16 = jnp.zeros((16,), jnp.float32)

    def zbody(i, _):
        deg_v[pl.ds(i * 16, 16)] = z16
        return 0

    lax.fori_loop(0, N_PAD // 16, zbody, 0)
    ones16 = jnp.ones((16,), jnp.float32)

    def body(j, _):
        for k in range(CHUNK // 16):
            idx16 = idx_v[j, pl.ds(k * 16, 16)]
            plsc.addupdate_scatter(deg_v, [idx16], ones16)
        return 0

    lax.fori_loop(0, CHUNKS, body, 0)
    pltpu.sync_copy(deg_v, out_hbm.at[wid])


@functools.partial(
    pl.kernel,
    out_type=jax.ShapeDtypeStruct((2, N_PAD, D), jnp.float32),
    mesh=_mesh,
    scratch_types=[
        pltpu.VMEM((CHUNKS, CHUNK), jnp.int32),
        pltpu.VMEM((CHUNKS, CHUNK), jnp.int32),
        pltpu.VMEM((CHUNK, D), jnp.float32),
        pltpu.VMEM_SHARED((N_PAD, D), jnp.float32),
        pltpu.SemaphoreType.DMA,
    ],
)
def _agg_kernel(xs_hbm, src_hbm, dst_hbm, zero_hbm, out_hbm,
                src_v, dst_v, rows0_v, acc_sh, sem0):
    cid = lax.axis_index("c")
    sid = lax.axis_index("s")
    wid = cid * TILES + sid
    pltpu.sync_copy(zero_hbm.at[pl.ds(sid * ROWS_PER_TILE, ROWS_PER_TILE)],
                    acc_sh.at[pl.ds(sid * ROWS_PER_TILE, ROWS_PER_TILE)])
    plsc.subcore_barrier()

    for q in range(CHUNKS // STAGE):
        pltpu.sync_copy(src_hbm.at[wid, q],
                        src_v.at[pl.ds(q * STAGE, STAGE)])
        pltpu.sync_copy(dst_hbm.at[wid, q],
                        dst_v.at[pl.ds(q * STAGE, STAGE)])

    def body(j, _):
        pltpu.async_copy(xs_hbm.at[src_v.at[j]], rows0_v, sem0).wait()
        pltpu.sync_copy(rows0_v, acc_sh.at[dst_v.at[j]], add=True)
        return 0

    lax.fori_loop(0, CHUNKS, body, 0)
    plsc.subcore_barrier()
    pltpu.sync_copy(acc_sh.at[pl.ds(sid * ROWS_PER_TILE, ROWS_PER_TILE)],
                    out_hbm.at[cid, pl.ds(sid * ROWS_PER_TILE, ROWS_PER_TILE)])


def _tc1_body(x_ref, w_ref, degp_ref, xs_ref, dinv_ref):
    deg = jnp.sum(degp_ref[...], axis=0) + 1.0
    dinv = lax.rsqrt(deg)
    dinv_ref[...] = dinv
    xw = jnp.dot(x_ref[...], w_ref[...], preferred_element_type=jnp.float32)
    xs_ref[...] = xw * dinv[:N_NODES, None]


def _tc2_body(p_ref, xs_ref, dinv_ref, b_ref, w_ref, xs2_ref):
    dinv = dinv_ref[...][:N_NODES, None]
    acc = p_ref[0, :N_NODES, :] + p_ref[1, :N_NODES, :] + xs_ref[...]
    h = jnp.maximum(dinv * acc + b_ref[...][None, :], 0.0)
    xs2_ref[...] = jnp.dot(h, w_ref[...], preferred_element_type=jnp.float32) * dinv


def _tc3_body(p_ref, xs_ref, dinv_ref, b_ref, out_ref):
    dinv = dinv_ref[...][:N_NODES, None]
    acc = p_ref[0, :N_NODES, :] + p_ref[1, :N_NODES, :] + xs_ref[...]
    out_ref[...] = dinv * acc + b_ref[...][None, :]


def _tc_call(body, out_shape, *args):
    return pl.pallas_call(body, out_shape=out_shape)(*args)


def kernel(x, edge_index, W1, b1, W2, b2):
    ei = edge_index.astype(jnp.int32)
    pad = E_PAD - E
    fill = jnp.arange(pad, dtype=jnp.int32) % N_NODES
    src = jnp.concatenate([ei[0], fill])
    junk = N_NODES + (jnp.arange(pad, dtype=jnp.int32) % (N_PAD - N_NODES))
    dst = jnp.concatenate([ei[1], junk])
    src3 = src.reshape(NW, CHUNKS // STAGE, STAGE, CHUNK)
    dst3 = dst.reshape(NW, CHUNKS // STAGE, STAGE, CHUNK)
    zeros = jnp.zeros((N_PAD, D), jnp.float32)

    degp = _deg_kernel(dst3)
    xs1, dinv = _tc_call(
        _tc1_body,
        (jax.ShapeDtypeStruct((N_NODES, D), jnp.float32),
         jax.ShapeDtypeStruct((N_PAD,), jnp.float32)),
        x, W1, degp)
    parts1 = _agg_kernel(xs1, src3, dst3, zeros)
    xs2 = _tc_call(
        _tc2_body,
        jax.ShapeDtypeStruct((N_NODES, D), jnp.float32),
        parts1, xs1, dinv, b1, W2)
    parts2 = _agg_kernel(xs2, src3, dst3, zeros)
    out = _tc_call(
        _tc3_body,
        jax.ShapeDtypeStruct((N_NODES, D), jnp.float32),
        parts2, xs2, dinv, b2)
    return out


# speedup vs baseline: 2.8013x; 1.1355x over previous
import functools
import jax
import jax.numpy as jnp
from jax import lax
from jax.experimental import pallas as pl
from jax.experimental.pallas import tpu as pltpu
from jax.experimental.pallas import tpu_sc as plsc

N_NODES = 10000
D = 128
E = 320000
NW = 32
TILES = 16
CHUNK = 128
CHUNKS = 80
STAGE = 40
E_PAD = NW * CHUNKS * CHUNK
N_PAD = 10240
ROWS_PER_TILE = N_PAD // TILES

_mesh = plsc.VectorSubcoreMesh(core_axis_name="c", subcore_axis_name="s")


@functools.partial(
    pl.kernel,
    out_type=jax.ShapeDtypeStruct((NW, N_PAD), jnp.float32),
    mesh=_mesh,
    compiler_params=pltpu.CompilerParams(needs_layout_passes=False),
    scratch_types=[
        pltpu.VMEM((CHUNKS, CHUNK), jnp.int32),
        pltpu.VMEM((N_PAD,), jnp.float32),
    ],
)
def _deg_kernel(dst_hbm, out_hbm, idx_v, deg_v):
    cid = lax.axis_index("c")
    sid = lax.axis_index("s")
    wid = cid * TILES + sid
    for q in range(CHUNKS // STAGE):
        pltpu.sync_copy(dst_hbm.at[wid, q],
                        idx_v.at[pl.ds(q * STAGE, STAGE)])
    z16 = jnp.zeros((16,), jnp.float32)

    def zbody(i, _):
        deg_v[pl.ds(i * 16, 16)] = z16
        return 0

    lax.fori_loop(0, N_PAD // 16, zbody, 0)
    ones16 = jnp.ones((16,), jnp.float32)

    def body(j, _):
        for k in range(CHUNK // 16):
            idx16 = idx_v[j, pl.ds(k * 16, 16)]
            plsc.addupdate_scatter(deg_v, [idx16], ones16)
        return 0

    lax.fori_loop(0, CHUNKS, body, 0)
    pltpu.sync_copy(deg_v, out_hbm.at[wid])


@functools.partial(
    pl.kernel,
    out_type=jax.ShapeDtypeStruct((2, N_PAD, D), jnp.float32),
    mesh=_mesh,
    scratch_types=[
        pltpu.VMEM((STAGE, CHUNK), jnp.int32),
        pltpu.VMEM((STAGE, CHUNK), jnp.int32),
        pltpu.VMEM((CHUNK, D), jnp.float32),
        pltpu.VMEM((CHUNK, D), jnp.float32),
        pltpu.VMEM_SHARED((N_PAD, D), jnp.float32),
        pltpu.SemaphoreType.DMA,
        pltpu.SemaphoreType.DMA,
    ],
)
def _agg_kernel(xs_hbm, src_hbm, dst_hbm, zero_hbm, out_hbm,
                src_v, dst_v, rows0_v, rows1_v, acc_sh, sem0, sem1):
    cid = lax.axis_index("c")
    sid = lax.axis_index("s")
    wid = cid * TILES + sid
    pltpu.sync_copy(zero_hbm.at[pl.ds(sid * ROWS_PER_TILE, ROWS_PER_TILE)],
                    acc_sh.at[pl.ds(sid * ROWS_PER_TILE, ROWS_PER_TILE)])
    plsc.subcore_barrier()

    for q in range(CHUNKS // STAGE):
        pltpu.sync_copy(src_hbm.at[wid, q], src_v)
        pltpu.sync_copy(dst_hbm.at[wid, q], dst_v)

        def body(g, _):
            j0 = 2 * g
            d0 = pltpu.async_copy(xs_hbm.at[src_v.at[j0]], rows0_v, sem0)
            d1 = pltpu.async_copy(xs_hbm.at[src_v.at[j0 + 1]], rows1_v, sem1)
            d0.wait()
            pltpu.sync_copy(rows0_v, acc_sh.at[dst_v.at[j0]], add=True)
            d1.wait()
            pltpu.sync_copy(rows1_v, acc_sh.at[dst_v.at[j0 + 1]], add=True)
            return 0

        lax.fori_loop(0, STAGE // 2, body, 0)
    plsc.subcore_barrier()
    pltpu.sync_copy(acc_sh.at[pl.ds(sid * ROWS_PER_TILE, ROWS_PER_TILE)],
                    out_hbm.at[cid, pl.ds(sid * ROWS_PER_TILE, ROWS_PER_TILE)])


def _tc1_body(x_ref, w_ref, degp_ref, xs_ref, dinv_ref):
    deg = jnp.sum(degp_ref[...], axis=0) + 1.0
    dinv = lax.rsqrt(deg)
    dinv_ref[...] = dinv
    xw = jnp.dot(x_ref[...], w_ref[...], preferred_element_type=jnp.float32)
    xs_ref[...] = xw * dinv[:N_NODES, None]


def _tc2_body(p_ref, xs_ref, dinv_ref, b_ref, w_ref, xs2_ref):
    dinv = dinv_ref[...][:N_NODES, None]
    acc = p_ref[0, :N_NODES, :] + p_ref[1, :N_NODES, :] + xs_ref[...]
    h = jnp.maximum(dinv * acc + b_ref[...][None, :], 0.0)
    xs2_ref[...] = jnp.dot(h, w_ref[...], preferred_element_type=jnp.float32) * dinv


def _tc3_body(p_ref, xs_ref, dinv_ref, b_ref, out_ref):
    dinv = dinv_ref[...][:N_NODES, None]
    acc = p_ref[0, :N_NODES, :] + p_ref[1, :N_NODES, :] + xs_ref[...]
    out_ref[...] = dinv * acc + b_ref[...][None, :]


def _tc_call(body, out_shape, *args):
    return pl.pallas_call(body, out_shape=out_shape)(*args)


def kernel(x, edge_index, W1, b1, W2, b2):
    ei = edge_index.astype(jnp.int32)
    pad = E_PAD - E
    fill = jnp.arange(pad, dtype=jnp.int32) % N_NODES
    src = jnp.concatenate([ei[0], fill])
    junk = N_NODES + (jnp.arange(pad, dtype=jnp.int32) % (N_PAD - N_NODES))
    dst = jnp.concatenate([ei[1], junk])
    src3 = src.reshape(NW, CHUNKS // STAGE, STAGE, CHUNK)
    dst3 = dst.reshape(NW, CHUNKS // STAGE, STAGE, CHUNK)
    zeros = jnp.zeros((N_PAD, D), jnp.float32)

    degp = _deg_kernel(dst3)
    xs1, dinv = _tc_call(
        _tc1_body,
        (jax.ShapeDtypeStruct((N_NODES, D), jnp.float32),
         jax.ShapeDtypeStruct((N_PAD,), jnp.float32)),
        x, W1, degp)
    parts1 = _agg_kernel(xs1, src3, dst3, zeros)
    xs2 = _tc_call(
        _tc2_body,
        jax.ShapeDtypeStruct((N_NODES, D), jnp.float32),
        parts1, xs1, dinv, b1, W2)
    parts2 = _agg_kernel(xs2, src3, dst3, zeros)
    out = _tc_call(
        _tc3_body,
        jax.ShapeDtypeStruct((N_NODES, D), jnp.float32),
        parts2, xs2, dinv, b2)
    return out
